# baseline (speedup 1.0000x reference)
import jax
import jax.numpy as jnp
from jax.experimental import pallas as pl

_CONTRACT_ROWS = (((0,), (0,)), ((), ()))


def _readout_body(z_ref, hc_ref, hd_ref, case_ref, dis_ref):
    z = z_ref[...].astype(jnp.bfloat16)
    hc = hc_ref[...]
    case_mm = jax.lax.dot_general(
        hc.astype(jnp.bfloat16), z, _CONTRACT_ROWS,
        preferred_element_type=jnp.float32,
    )
    deg = jnp.clip(jnp.sum(hc, axis=0), 1e-6, None)
    case_ref[...] = case_mm / deg[:, None]
    dis_ref[...] = jax.lax.dot_general(
        hd_ref[...].astype(jnp.bfloat16), z, _CONTRACT_ROWS,
        preferred_element_type=jnp.float32,
    )


def kernel(Z, H_case, H_disease):
    n, d = Z.shape
    e = H_case.shape[1]
    tile_e = 128
    grid = (e // tile_e,)
    case_repr, disease_repr = pl.pallas_call(
        _readout_body,
        grid=grid,
        in_specs=[
            pl.BlockSpec((n, d), lambda j: (0, 0)),
            pl.BlockSpec((n, tile_e), lambda j: (0, j)),
            pl.BlockSpec((n, tile_e), lambda j: (0, j)),
        ],
        out_specs=[
            pl.BlockSpec((tile_e, d), lambda j: (j, 0)),
            pl.BlockSpec((tile_e, d), lambda j: (j, 0)),
        ],
        out_shape=[
            jax.ShapeDtypeStruct((e, d), jnp.float32),
            jax.ShapeDtypeStruct((e, d), jnp.float32),
        ],
    )(Z, H_case, H_disease)
    return (case_repr, disease_repr)


# speedup vs baseline: 1.1472x; 1.1472x over previous
import jax
import jax.numpy as jnp
from jax.experimental import pallas as pl
from jax.experimental.pallas import tpu as pltpu

_CONTRACT_ROWS = (((0,), (0,)), ((), ()))


def _readout_body(z_ref, hc_ref, hd_ref, case_ref, dis_ref, deg_ref):
    i = pl.program_id(0)
    z = z_ref[...].astype(jnp.bfloat16)
    hc = hc_ref[...]
    cm = jax.lax.dot_general(
        hc.astype(jnp.bfloat16), z, _CONTRACT_ROWS,
        preferred_element_type=jnp.float32,
    )
    dm = jax.lax.dot_general(
        hd_ref[...].astype(jnp.bfloat16), z, _CONTRACT_ROWS,
        preferred_element_type=jnp.float32,
    )
    degp = jnp.sum(hc, axis=0)

    @pl.when(i == 0)
    def _init():
        case_ref[...] = cm
        dis_ref[...] = dm
        deg_ref[...] = degp

    @pl.when(i > 0)
    def _acc():
        case_ref[...] += cm
        dis_ref[...] += dm
        deg_ref[...] += degp

    @pl.when(i == pl.num_programs(0) - 1)
    def _fin():
        deg = jnp.clip(deg_ref[...], 1e-6, None)
        case_ref[...] = case_ref[...] / deg[:, None]


def kernel(Z, H_case, H_disease):
    n, d = Z.shape
    e = H_case.shape[1]
    tile_n = 512
    grid = (n // tile_n,)
    case_repr, disease_repr = pl.pallas_call(
        _readout_body,
        grid=grid,
        in_specs=[
            pl.BlockSpec((tile_n, d), lambda i: (i, 0)),
            pl.BlockSpec((tile_n, e), lambda i: (i, 0)),
            pl.BlockSpec((tile_n, e), lambda i: (i, 0)),
        ],
        out_specs=[
            pl.BlockSpec((e, d), lambda i: (0, 0)),
            pl.BlockSpec((e, d), lambda i: (0, 0)),
        ],
        out_shape=[
            jax.ShapeDtypeStruct((e, d), jnp.float32),
            jax.ShapeDtypeStruct((e, d), jnp.float32),
        ],
        scratch_shapes=[pltpu.VMEM((e,), jnp.float32)],
    )(Z, H_case, H_disease)
    return (case_repr, disease_repr)
